# Initial kernel scaffold; baseline (speedup 1.0000x reference)
import jax
import jax.numpy as jnp
from jax.experimental import pallas as pl


def kernel(positions, species, senders, receivers, n_node, target_species, species_embed, node_embed, W_r1, b_r1, W_r2, W_self, W_focus, b_focus, W_species, b_species, W_pos, b_pos):
    raise NotImplementedError("write your pallas kernel here")


# speedup vs baseline: 12.8492x; 12.8492x over previous
import jax
import jax.numpy as jnp
from jax.experimental import pallas as pl
from jax.experimental.pallas import tpu as pltpu

N = 10000
E = 320000
LAT = 128
NRB = 8
RMAX = 2.0
C1 = 32
C2 = 16
AVG_NEIGH = 32.0
N_RADII = 64
IRREPS_DIM = 36
D_OUT = LAT + C1 * 3 + C2 * 5

BE = 2000


def _silu(x):
    return x * jax.nn.sigmoid(x)


def _edge_kernel(vec_ref, sf_ref, wr1_ref, br1_ref, wr2_ref, m_ref):
    vec = vec_ref[...]
    sf = sf_ref[...]
    d2 = jnp.sum(vec * vec, axis=-1, keepdims=True)
    d = jnp.sqrt(d2)
    inv_d = 1.0 / (d + 1e-9)
    u = vec * inv_d
    n = (jnp.arange(1, NRB + 1, dtype=jnp.int32)
         .astype(jnp.float32))[None, :]
    rbf = jnp.sqrt(2.0 / RMAX) * jnp.sin(n * (jnp.pi / RMAX) * d) * inv_d
    env = 0.5 * (jnp.cos(jnp.pi * jnp.clip(d / RMAX, 0.0, 1.0)) + 1.0)
    rbf = rbf * env
    h = _silu(jnp.dot(rbf, wr1_ref[...], preferred_element_type=jnp.float32)
              + br1_ref[...])
    w = jnp.dot(h, wr2_ref[...], preferred_element_type=jnp.float32)
    w0 = w[:, :LAT]
    w1 = w[:, LAT:LAT + C1]
    w2 = w[:, LAT + C1:]
    x, y, z = u[:, 0:1], u[:, 1:2], u[:, 2:3]
    s3 = jnp.sqrt(3.0)
    sh2_list = [s3 * x * y, s3 * y * z, 0.5 * (3.0 * z * z - 1.0),
                s3 * x * z, 0.5 * s3 * (x * x - y * y)]
    t1 = sf[:, :C1] * w1
    t2 = sf[:, :C2] * w2
    m0 = sf * w0
    parts = [m0]
    for k in range(3):
        parts.append(u[:, k:k + 1] * t1)
    for k in range(5):
        parts.append(sh2_list[k] * t2)
    m_ref[...] = jnp.concatenate(parts, axis=-1)


def _edge_messages(vec, sf, W_r1, b_r1, W_r2):
    grid = (E // BE,)
    return pl.pallas_call(
        _edge_kernel,
        grid=grid,
        in_specs=[
            pl.BlockSpec((BE, 3), lambda i: (i, 0)),
            pl.BlockSpec((BE, LAT), lambda i: (i, 0)),
            pl.BlockSpec((NRB, 64), lambda i: (0, 0)),
            pl.BlockSpec((64,), lambda i: (0,)),
            pl.BlockSpec((64, LAT + C1 + C2), lambda i: (0, 0)),
        ],
        out_specs=pl.BlockSpec((BE, D_OUT), lambda i: (i, 0)),
        out_shape=jax.ShapeDtypeStruct((E, D_OUT), jnp.float32),
    )(vec, sf, W_r1, b_r1, W_r2)


def kernel(positions, species, senders, receivers, n_node, target_species,
           species_embed, node_embed, W_r1, b_r1, W_r2, W_self,
           W_focus, b_focus, W_species, b_species, W_pos, b_pos):
    vec = positions[receivers] - positions[senders]
    nf = node_embed[species]
    sf = nf[senders]
    M = _edge_messages(vec, sf, W_r1, b_r1, W_r2)
    A = jax.ops.segment_sum(M, receivers, num_segments=N) * (1.0 / jnp.sqrt(AVG_NEIGH))
    a0 = A[:, :LAT]
    a1b = A[:, LAT:LAT + 3 * C1]
    a2b = A[:, LAT + 3 * C1:]
    a1 = a1b.reshape(N, 3, C1).transpose(0, 2, 1)
    a2 = a2b.reshape(N, 5, C2).transpose(0, 2, 1)
    scal = _silu(a0 + nf @ W_self)
    g1 = jax.nn.sigmoid(scal[:, :C1])
    g2 = jax.nn.sigmoid(scal[:, C1:C1 + C2])
    v = (a1 * g1[:, :, None]).reshape(N, C1 * 3)
    t = (a2 * g2[:, :, None]).reshape(N, C2 * 5)
    node_embeddings = jnp.concatenate([scal, v, t], axis=-1)
    first_idx = jnp.concatenate(
        [jnp.asarray([0], dtype=n_node.dtype), jnp.cumsum(n_node)[:-1]])
    focus_logits = (node_embeddings @ W_focus + b_focus)[:, 0]
    true_focus = node_embeddings[first_idx]
    species_logits = true_focus @ W_species + b_species
    ts_emb = species_embed[target_species]
    pos_in = jnp.concatenate([true_focus, ts_emb], axis=-1)
    position_coeffs = (pos_in @ W_pos + b_pos).reshape(-1, N_RADII, IRREPS_DIM)
    return focus_logits, species_logits, position_coeffs


# speedup vs baseline: 14.7531x; 1.1482x over previous
import functools

import jax
import jax.numpy as jnp
from jax import lax
from jax.experimental import pallas as pl
from jax.experimental.pallas import tpu as pltpu
from jax.experimental.pallas import tpu_sc as plsc

N = 10000
E = 320000
LAT = 128
NRB = 8
RMAX = 2.0
C1 = 32
C2 = 16
AVG_NEIGH = 32.0
N_RADII = 64
IRREPS_DIM = 36
D_OUT = LAT + C1 * 3 + C2 * 5

BE = 2000


def _silu(x):
    return x * jax.nn.sigmoid(x)


def _edge_kernel(vec_ref, sf_ref, wr1_ref, br1_ref, wr2_ref, m_ref):
    vec = vec_ref[...]
    sf = sf_ref[...]
    d2 = jnp.sum(vec * vec, axis=-1, keepdims=True)
    d = jnp.sqrt(d2)
    inv_d = 1.0 / (d + 1e-9)
    u = vec * inv_d
    n = (jnp.arange(1, NRB + 1, dtype=jnp.int32)
         .astype(jnp.float32))[None, :]
    rbf = jnp.sqrt(2.0 / RMAX) * jnp.sin(n * (jnp.pi / RMAX) * d) * inv_d
    env = 0.5 * (jnp.cos(jnp.pi * jnp.clip(d / RMAX, 0.0, 1.0)) + 1.0)
    rbf = rbf * env
    h = _silu(jnp.dot(rbf, wr1_ref[...], preferred_element_type=jnp.float32)
              + br1_ref[...])
    w = jnp.dot(h, wr2_ref[...], preferred_element_type=jnp.float32)
    w0 = w[:, :LAT]
    w1 = w[:, LAT:LAT + C1]
    w2 = w[:, LAT + C1:]
    x, y, z = u[:, 0:1], u[:, 1:2], u[:, 2:3]
    s3 = jnp.sqrt(3.0)
    sh2_list = [s3 * x * y, s3 * y * z, 0.5 * (3.0 * z * z - 1.0),
                s3 * x * z, 0.5 * s3 * (x * x - y * y)]
    t1 = sf[:, :C1] * w1
    t2 = sf[:, :C2] * w2
    m0 = sf * w0
    p1 = [u[:, k:k + 1] * t1 for k in range(3)]
    p1 += [sh2_list[k] * t2 for k in range(2)]
    p2 = [sh2_list[k] * t2 for k in range(2, 5)]
    p2.append(jnp.zeros((BE, 128 - 3 * C2), jnp.float32))
    m_ref[0] = m0
    m_ref[1] = jnp.concatenate(p1, axis=-1)
    m_ref[2] = jnp.concatenate(p2, axis=-1)


def _edge_messages(vec, sf, W_r1, b_r1, W_r2):
    grid = (E // BE,)
    return pl.pallas_call(
        _edge_kernel,
        grid=grid,
        in_specs=[
            pl.BlockSpec((BE, 3), lambda i: (i, 0)),
            pl.BlockSpec((BE, LAT), lambda i: (i, 0)),
            pl.BlockSpec((NRB, 64), lambda i: (0, 0)),
            pl.BlockSpec((64,), lambda i: (0,)),
            pl.BlockSpec((64, LAT + C1 + C2), lambda i: (0, 0)),
        ],
        out_specs=pl.BlockSpec((3, BE, 128), lambda i: (0, i, 0)),
        out_shape=jax.ShapeDtypeStruct((3, E, 128), jnp.float32),
    )(vec, sf, W_r1, b_r1, W_r2)


CG = 40
NROW = 10240


def _sc_scatter_body(mp_hbm, recv2_hbm, out_hbm, mbuf, ibuf, acc, sem, sem2):
    c = lax.axis_index("c")
    s = lax.axis_index("s")
    zeros = jnp.zeros((16,), jnp.float32)

    def zero_mbuf(rr, _):
        for f in range(8):
            mbuf[rr, pl.ds(f * 16, 16)] = zeros
        return 0

    def zero_acc():
        lax.fori_loop(0, CG, zero_mbuf, 0)

        def zd(q, _):
            pltpu.sync_copy(mbuf, acc.at[pl.ds(s * 640 + q * CG, CG)])
            return 0

        lax.fori_loop(0, 640 // CG, zd, 0)

    def run_phase(pidx, grow0, ngrp):
        def group(g, _):
            pltpu.sync_copy(recv2_hbm.at[pl.ds((grow0 + g) * 16, 16)], ibuf)

            def chunk(j, _):
                e0 = ((grow0 + g) * 16 + j) * CG
                pltpu.async_copy(mp_hbm.at[pidx, pl.ds(e0, CG)], mbuf,
                                 sem).wait()
                pltpu.async_copy(mbuf, acc.at[ibuf.at[j]], sem2,
                                 add=True).wait()
                return 0

            lax.fori_loop(0, 16, chunk, 0)
            return 0

        lax.fori_loop(0, ngrp, group, 0)

    def writeout(oidx):
        pltpu.sync_copy(acc.at[pl.ds(s * 640, 640)],
                        out_hbm.at[oidx, pl.ds(s * 640, 640)])

    zero_acc()
    plsc.subcore_barrier()
    gA = 31 * s + jnp.minimum(s, 4)
    nA = jnp.where(s < 4, 32, 31)
    run_phase(c, gA, nA)
    plsc.subcore_barrier()
    writeout(c)
    plsc.subcore_barrier()
    zero_acc()
    plsc.subcore_barrier()
    gB = c * 250 + 15 * s + jnp.minimum(s, 10)
    nB = jnp.where(s < 10, 16, 15)
    run_phase(2, gB, nB)
    plsc.subcore_barrier()
    writeout(2 + c)


def _sc_segment_sum(Mp, receivers):
    mesh = plsc.VectorSubcoreMesh(core_axis_name="c", subcore_axis_name="s")
    f = pl.kernel(
        _sc_scatter_body,
        out_type=jax.ShapeDtypeStruct((4, NROW, 128), jnp.float32),
        mesh=mesh,
        scratch_types=[
            pltpu.VMEM((CG, 128), jnp.float32),
            pltpu.VMEM((16, CG), jnp.int32),
            pltpu.VMEM_SHARED((NROW, 128), jnp.float32),
            pltpu.SemaphoreType.DMA,
            pltpu.SemaphoreType.DMA,
        ],
    )
    out = f(Mp, receivers.reshape(E // CG, CG))
    a2 = out[2, :N, :48] + out[3, :N, :48]
    return jnp.concatenate([out[0, :N], out[1, :N], a2], axis=-1)


def kernel(positions, species, senders, receivers, n_node, target_species,
           species_embed, node_embed, W_r1, b_r1, W_r2, W_self,
           W_focus, b_focus, W_species, b_species, W_pos, b_pos):
    vec = positions[receivers] - positions[senders]
    nf = node_embed[species]
    sf = nf[senders]
    M = _edge_messages(vec, sf, W_r1, b_r1, W_r2)
    A = _sc_segment_sum(M, receivers) * (1.0 / jnp.sqrt(AVG_NEIGH))
    a0 = A[:, :LAT]
    a1b = A[:, LAT:LAT + 3 * C1]
    a2b = A[:, LAT + 3 * C1:]
    a1 = a1b.reshape(N, 3, C1).transpose(0, 2, 1)
    a2 = a2b.reshape(N, 5, C2).transpose(0, 2, 1)
    scal = _silu(a0 + nf @ W_self)
    g1 = jax.nn.sigmoid(scal[:, :C1])
    g2 = jax.nn.sigmoid(scal[:, C1:C1 + C2])
    v = (a1 * g1[:, :, None]).reshape(N, C1 * 3)
    t = (a2 * g2[:, :, None]).reshape(N, C2 * 5)
    node_embeddings = jnp.concatenate([scal, v, t], axis=-1)
    first_idx = jnp.concatenate(
        [jnp.asarray([0], dtype=n_node.dtype), jnp.cumsum(n_node)[:-1]])
    focus_logits = (node_embeddings @ W_focus + b_focus)[:, 0]
    true_focus = node_embeddings[first_idx]
    species_logits = true_focus @ W_species + b_species
    ts_emb = species_embed[target_species]
    pos_in = jnp.concatenate([true_focus, ts_emb], axis=-1)
    position_coeffs = (pos_in @ W_pos + b_pos).reshape(-1, N_RADII, IRREPS_DIM)
    return focus_logits, species_logits, position_coeffs


# speedup vs baseline: 18.9261x; 1.2829x over previous
import functools

import jax
import jax.numpy as jnp
from jax import lax
from jax.experimental import pallas as pl
from jax.experimental.pallas import tpu as pltpu
from jax.experimental.pallas import tpu_sc as plsc

N = 10000
E = 320000
LAT = 128
NRB = 8
RMAX = 2.0
C1 = 32
C2 = 16
AVG_NEIGH = 32.0
N_RADII = 64
IRREPS_DIM = 36
D_OUT = LAT + C1 * 3 + C2 * 5

BE = 2560


def _silu(x):
    return x * jax.nn.sigmoid(x)


import numpy as _np

_W1P_PAD = 16
_P1 = _np.zeros((16, 128), _np.float32)
for _k in range(3):
    _P1[8 + _k, 32 * _k:32 * (_k + 1)] = 1.0
for _k in range(2):
    _P1[11 + _k, 96 + 16 * _k:96 + 16 * (_k + 1)] = 1.0
_P2 = _np.zeros((16, 128), _np.float32)
for _k in range(3):
    _P2[13 + _k, 16 * _k:16 * (_k + 1)] = 1.0
_Q = _np.zeros((128, 256), _np.float32)
for _k in range(3):
    for _j in range(32):
        _Q[_j, 32 * _k + _j] = 1.0
for _k in range(2):
    for _j in range(16):
        _Q[_j, 96 + 16 * _k + _j] = 1.0
for _k in range(3):
    for _j in range(16):
        _Q[_j, 128 + 16 * _k + _j] = 1.0


def _expand_weights(W_r1, W_r2):
    W1p = jnp.zeros((16, 64), jnp.float32).at[:NRB].set(W_r1)
    w1 = W_r2[:, LAT:LAT + C1]
    w2 = W_r2[:, LAT + C1:]
    W2exp = jnp.concatenate(
        [W_r2[:, :LAT], w1, w1, w1, w2, w2, w2, w2, w2,
         jnp.zeros((64, 80), jnp.float32)], axis=1)
    return W1p, W2exp


def _edge_kernel(vecT_ref, sf_ref, w1p_ref, br1_ref, w2e_ref, p1_ref,
                 p2_ref, q_ref, m_ref):
    vx = vecT_ref[0, :]
    vy = vecT_ref[1, :]
    vz = vecT_ref[2, :]
    sf = sf_ref[...]
    d2 = vx * vx + vy * vy + vz * vz
    d = jnp.sqrt(d2)
    inv = 1.0 / (d + 1e-9)
    th = (jnp.pi / RMAX) * d
    s1 = jnp.sin(th)
    c1 = jnp.cos(th)
    env = jnp.where(d <= RMAX, 0.5 * (c1 + 1.0), 0.0)
    g = jnp.sqrt(2.0 / RMAX) * env * inv
    two_c1 = 2.0 * c1
    cols = [s1 * g]
    s_prev, s_cur = jnp.zeros_like(s1), s1
    for _ in range(NRB - 1):
        s_prev, s_cur = s_cur, two_c1 * s_cur - s_prev
        cols.append(s_cur * g)
    ux = vx * inv
    uy = vy * inv
    uz = vz * inv
    s3 = jnp.sqrt(3.0)
    cols += [ux, uy, uz,
             s3 * ux * uy, s3 * uy * uz, 0.5 * (3.0 * uz * uz - 1.0),
             s3 * ux * uz, 0.5 * s3 * (ux * ux - uy * uy)]
    RU = jnp.stack(cols, axis=-1)
    h = _silu(jnp.dot(RU, w1p_ref[...], preferred_element_type=jnp.float32)
              + br1_ref[...])
    wb = jnp.dot(h, w2e_ref[...], preferred_element_type=jnp.float32)
    S1 = jnp.dot(RU, p1_ref[...], preferred_element_type=jnp.float32)
    S2 = jnp.dot(RU, p2_ref[...], preferred_element_type=jnp.float32)
    sft = jnp.dot(sf, q_ref[...], preferred_element_type=jnp.float32)
    m_ref[0] = sf * wb[:, :128]
    m_ref[1] = S1 * sft[:, :128] * wb[:, 128:256]
    m_ref[2] = S2 * sft[:, 128:256] * wb[:, 256:384]


def _edge_messages(vecT, sf, W_r1, b_r1, W_r2):
    W1p, W2exp = _expand_weights(W_r1, W_r2)
    grid = (E // BE,)
    return pl.pallas_call(
        _edge_kernel,
        grid=grid,
        in_specs=[
            pl.BlockSpec((3, BE), lambda i: (0, i)),
            pl.BlockSpec((BE, LAT), lambda i: (i, 0)),
            pl.BlockSpec((16, 64), lambda i: (0, 0)),
            pl.BlockSpec((64,), lambda i: (0,)),
            pl.BlockSpec((64, 384), lambda i: (0, 0)),
            pl.BlockSpec((16, 128), lambda i: (0, 0)),
            pl.BlockSpec((16, 128), lambda i: (0, 0)),
            pl.BlockSpec((128, 256), lambda i: (0, 0)),
        ],
        out_specs=pl.BlockSpec((3, BE, 128), lambda i: (0, i, 0)),
        out_shape=jax.ShapeDtypeStruct((3, E, 128), jnp.float32),
    )(vecT, sf, W1p, b_r1, W2exp, jnp.asarray(_P1), jnp.asarray(_P2),
      jnp.asarray(_Q))


CG = 40
NROW = 10240


def _sc_scatter_body(mp_hbm, recv2_hbm, out_hbm, mbuf, ibuf, acc, sem, sem2):
    c = lax.axis_index("c")
    s = lax.axis_index("s")
    zeros = jnp.zeros((16,), jnp.float32)

    def zero_mbuf(rr, _):
        for f in range(8):
            mbuf[rr, pl.ds(f * 16, 16)] = zeros
        return 0

    def zero_acc():
        lax.fori_loop(0, CG, zero_mbuf, 0)

        def zd(q, _):
            pltpu.sync_copy(mbuf, acc.at[pl.ds(s * 640 + q * CG, CG)])
            return 0

        lax.fori_loop(0, 640 // CG, zd, 0)

    def run_phase(pidx, grow0, ngrp):
        def group(g, _):
            pltpu.sync_copy(recv2_hbm.at[pl.ds((grow0 + g) * 16, 16)], ibuf)

            def chunk(j, _):
                e0 = ((grow0 + g) * 16 + j) * CG
                pltpu.async_copy(mp_hbm.at[pidx, pl.ds(e0, CG)], mbuf,
                                 sem).wait()
                pltpu.async_copy(mbuf, acc.at[ibuf.at[j]], sem2,
                                 add=True).wait()
                return 0

            lax.fori_loop(0, 16, chunk, 0)
            return 0

        lax.fori_loop(0, ngrp, group, 0)

    def writeout(oidx):
        pltpu.sync_copy(acc.at[pl.ds(s * 640, 640)],
                        out_hbm.at[oidx, pl.ds(s * 640, 640)])

    zero_acc()
    plsc.subcore_barrier()
    gA = 31 * s + jnp.minimum(s, 4)
    nA = jnp.where(s < 4, 32, 31)
    run_phase(c, gA, nA)
    plsc.subcore_barrier()
    writeout(c)
    plsc.subcore_barrier()
    zero_acc()
    plsc.subcore_barrier()
    gB = c * 250 + 15 * s + jnp.minimum(s, 10)
    nB = jnp.where(s < 10, 16, 15)
    run_phase(2, gB, nB)
    plsc.subcore_barrier()
    writeout(2 + c)


def _sc_segment_sum(Mp, receivers):
    mesh = plsc.VectorSubcoreMesh(core_axis_name="c", subcore_axis_name="s")
    f = pl.kernel(
        _sc_scatter_body,
        out_type=jax.ShapeDtypeStruct((4, NROW, 128), jnp.float32),
        mesh=mesh,
        scratch_types=[
            pltpu.VMEM((CG, 128), jnp.float32),
            pltpu.VMEM((16, CG), jnp.int32),
            pltpu.VMEM_SHARED((NROW, 128), jnp.float32),
            pltpu.SemaphoreType.DMA,
            pltpu.SemaphoreType.DMA,
        ],
    )
    out = f(Mp, receivers.reshape(E // CG, CG))
    a2 = out[2, :N, :48] + out[3, :N, :48]
    return jnp.concatenate([out[0, :N], out[1, :N], a2], axis=-1)


def kernel(positions, species, senders, receivers, n_node, target_species,
           species_embed, node_embed, W_r1, b_r1, W_r2, W_self,
           W_focus, b_focus, W_species, b_species, W_pos, b_pos):
    vecT = (positions[receivers] - positions[senders]).T
    nf = node_embed[species]
    sf = nf[senders]
    M = _edge_messages(vecT, sf, W_r1, b_r1, W_r2)
    A = _sc_segment_sum(M, receivers) * (1.0 / jnp.sqrt(AVG_NEIGH))
    a0 = A[:, :LAT]
    a1b = A[:, LAT:LAT + 3 * C1]
    a2b = A[:, LAT + 3 * C1:]
    a1 = a1b.reshape(N, 3, C1).transpose(0, 2, 1)
    a2 = a2b.reshape(N, 5, C2).transpose(0, 2, 1)
    scal = _silu(a0 + nf @ W_self)
    g1 = jax.nn.sigmoid(scal[:, :C1])
    g2 = jax.nn.sigmoid(scal[:, C1:C1 + C2])
    v = (a1 * g1[:, :, None]).reshape(N, C1 * 3)
    t = (a2 * g2[:, :, None]).reshape(N, C2 * 5)
    node_embeddings = jnp.concatenate([scal, v, t], axis=-1)
    first_idx = jnp.concatenate(
        [jnp.asarray([0], dtype=n_node.dtype), jnp.cumsum(n_node)[:-1]])
    focus_logits = (node_embeddings @ W_focus + b_focus)[:, 0]
    true_focus = node_embeddings[first_idx]
    species_logits = true_focus @ W_species + b_species
    ts_emb = species_embed[target_species]
    pos_in = jnp.concatenate([true_focus, ts_emb], axis=-1)
    position_coeffs = (pos_in @ W_pos + b_pos).reshape(-1, N_RADII, IRREPS_DIM)
    return focus_logits, species_logits, position_coeffs


# speedup vs baseline: 21.4012x; 1.1308x over previous
import functools

import jax
import jax.numpy as jnp
from jax import lax
from jax.experimental import pallas as pl
from jax.experimental.pallas import tpu as pltpu
from jax.experimental.pallas import tpu_sc as plsc

N = 10000
E = 320000
LAT = 128
NRB = 8
RMAX = 2.0
C1 = 32
C2 = 16
AVG_NEIGH = 32.0
N_RADII = 64
IRREPS_DIM = 36
D_OUT = LAT + C1 * 3 + C2 * 5

BE = 2560


def _silu(x):
    return x * jax.nn.sigmoid(x)


import numpy as _np

_W1P_PAD = 16
_P1 = _np.zeros((16, 128), _np.float32)
for _k in range(3):
    _P1[8 + _k, 32 * _k:32 * (_k + 1)] = 1.0
for _k in range(2):
    _P1[11 + _k, 96 + 16 * _k:96 + 16 * (_k + 1)] = 1.0
_P2 = _np.zeros((16, 128), _np.float32)
for _k in range(3):
    _P2[13 + _k, 16 * _k:16 * (_k + 1)] = 1.0
_Q = _np.zeros((128, 256), _np.float32)
for _k in range(3):
    for _j in range(32):
        _Q[_j, 32 * _k + _j] = 1.0
for _k in range(2):
    for _j in range(16):
        _Q[_j, 96 + 16 * _k + _j] = 1.0
for _k in range(3):
    for _j in range(16):
        _Q[_j, 128 + 16 * _k + _j] = 1.0


def _expand_weights(W_r1, W_r2):
    W1p = jnp.zeros((16, 64), jnp.float32).at[:NRB].set(W_r1)
    w1 = W_r2[:, LAT:LAT + C1]
    w2 = W_r2[:, LAT + C1:]
    W2exp = jnp.concatenate(
        [W_r2[:, :LAT], w1, w1, w1, w2, w2, w2, w2, w2,
         jnp.zeros((64, 80), jnp.float32)], axis=1)
    return W1p, W2exp


def _edge_kernel(vecT_ref, sf_ref, w1p_ref, br1_ref, w2e_ref, p1_ref,
                 p2_ref, q_ref, m_ref):
    vx = vecT_ref[0, :]
    vy = vecT_ref[1, :]
    vz = vecT_ref[2, :]
    sf = sf_ref[...]
    d2 = vx * vx + vy * vy + vz * vz
    d = jnp.sqrt(d2)
    inv = 1.0 / (d + 1e-9)
    th = (jnp.pi / RMAX) * d
    s1 = jnp.sin(th)
    c1 = jnp.cos(th)
    env = jnp.where(d <= RMAX, 0.5 * (c1 + 1.0), 0.0)
    g = jnp.sqrt(2.0 / RMAX) * env * inv
    two_c1 = 2.0 * c1
    cols = [s1 * g]
    s_prev, s_cur = jnp.zeros_like(s1), s1
    for _ in range(NRB - 1):
        s_prev, s_cur = s_cur, two_c1 * s_cur - s_prev
        cols.append(s_cur * g)
    ux = vx * inv
    uy = vy * inv
    uz = vz * inv
    s3 = jnp.sqrt(3.0)
    cols += [ux, uy, uz,
             s3 * ux * uy, s3 * uy * uz, 0.5 * (3.0 * uz * uz - 1.0),
             s3 * ux * uz, 0.5 * s3 * (ux * ux - uy * uy)]
    RU = jnp.stack(cols, axis=-1)
    h = _silu(jnp.dot(RU, w1p_ref[...], preferred_element_type=jnp.float32)
              + br1_ref[...])
    wb = jnp.dot(h, w2e_ref[...], preferred_element_type=jnp.float32)
    S1 = jnp.dot(RU, p1_ref[...], preferred_element_type=jnp.float32)
    S2 = jnp.dot(RU, p2_ref[...], preferred_element_type=jnp.float32)
    sft = jnp.dot(sf, q_ref[...], preferred_element_type=jnp.float32)
    m_ref[0] = sf * wb[:, :128]
    m_ref[1] = S1 * sft[:, :128] * wb[:, 128:256]
    m_ref[2] = S2 * sft[:, 128:256] * wb[:, 256:384]


def _edge_messages(vecT, sf, W_r1, b_r1, W_r2):
    W1p, W2exp = _expand_weights(W_r1, W_r2)
    grid = (E // BE,)
    return pl.pallas_call(
        _edge_kernel,
        grid=grid,
        in_specs=[
            pl.BlockSpec((3, BE), lambda i: (0, i)),
            pl.BlockSpec((BE, LAT), lambda i: (i, 0)),
            pl.BlockSpec((16, 64), lambda i: (0, 0)),
            pl.BlockSpec((64,), lambda i: (0,)),
            pl.BlockSpec((64, 384), lambda i: (0, 0)),
            pl.BlockSpec((16, 128), lambda i: (0, 0)),
            pl.BlockSpec((16, 128), lambda i: (0, 0)),
            pl.BlockSpec((128, 256), lambda i: (0, 0)),
        ],
        out_specs=pl.BlockSpec((3, BE, 128), lambda i: (0, i, 0)),
        out_shape=jax.ShapeDtypeStruct((3, E, 128), jnp.float32),
    )(vecT, sf, W1p, b_r1, W2exp, jnp.asarray(_P1), jnp.asarray(_P2),
      jnp.asarray(_Q))


def _sc_gather_body(emb_hbm, spc2_hbm, send2_hbm, nf_hbm, sf_hbm,
                    ibuf, gbuf, sem):
    c = lax.axis_index("c")
    s = lax.axis_index("s")

    pltpu.sync_copy(spc2_hbm.at[pl.ds(s * 16, 16)], ibuf)

    def nfrow(j, _):
        pltpu.async_copy(emb_hbm.at[ibuf.at[j]], gbuf, sem).wait()
        pltpu.sync_copy(gbuf, nf_hbm.at[pl.ds((s * 16 + j) * 40, 40)])
        return 0

    lax.fori_loop(0, 16, nfrow, 0)
    plsc.subcore_barrier()

    wid = s * 2 + c
    nblk = 31 + (wid < 8).astype(jnp.int32)

    def blk(i, _):
        r0 = (wid + 32 * i) * 8
        pltpu.sync_copy(send2_hbm.at[pl.ds(r0, 8)], ibuf.at[pl.ds(0, 8)])

        def row(j, _):
            pltpu.async_copy(nf_hbm.at[ibuf.at[j]], gbuf, sem).wait()
            pltpu.sync_copy(gbuf, sf_hbm.at[pl.ds((r0 + j) * 40, 40)])
            return 0

        lax.fori_loop(0, 8, row, 0)
        return 0

    lax.fori_loop(0, nblk, blk, 0)


def _sc_gather(node_embed, spc2, send2):
    mesh = plsc.VectorSubcoreMesh(core_axis_name="c", subcore_axis_name="s")
    f = pl.kernel(
        _sc_gather_body,
        out_type=(jax.ShapeDtypeStruct((NROW, LAT), jnp.float32),
                  jax.ShapeDtypeStruct((E, LAT), jnp.float32)),
        mesh=mesh,
        scratch_types=[
            pltpu.VMEM((16, 40), jnp.int32),
            pltpu.VMEM((40, LAT), jnp.float32),
            pltpu.SemaphoreType.DMA,
        ],
    )
    return f(node_embed, spc2, send2)


CG = 40
NROW = 10240


def _sc_scatter_body(mp_hbm, recv2_hbm, out_hbm, mbuf, ibuf, acc, sem, sem2):
    c = lax.axis_index("c")
    s = lax.axis_index("s")
    zeros = jnp.zeros((16,), jnp.float32)

    def zero_mbuf(rr, _):
        for f in range(8):
            mbuf[rr, pl.ds(f * 16, 16)] = zeros
        return 0

    def zero_acc():
        lax.fori_loop(0, CG, zero_mbuf, 0)

        def zd(q, _):
            pltpu.sync_copy(mbuf, acc.at[pl.ds(s * 640 + q * CG, CG)])
            return 0

        lax.fori_loop(0, 640 // CG, zd, 0)

    def run_phase(pidx, grow0, ngrp):
        def group(g, _):
            pltpu.sync_copy(recv2_hbm.at[pl.ds((grow0 + g) * 16, 16)], ibuf)

            def chunk(j, _):
                e0 = ((grow0 + g) * 16 + j) * CG
                pltpu.async_copy(mp_hbm.at[pidx, pl.ds(e0, CG)], mbuf,
                                 sem).wait()
                pltpu.async_copy(mbuf, acc.at[ibuf.at[j]], sem2,
                                 add=True).wait()
                return 0

            lax.fori_loop(0, 16, chunk, 0)
            return 0

        lax.fori_loop(0, ngrp, group, 0)

    def writeout(oidx):
        pltpu.sync_copy(acc.at[pl.ds(s * 640, 640)],
                        out_hbm.at[oidx, pl.ds(s * 640, 640)])

    zero_acc()
    plsc.subcore_barrier()
    gA = 31 * s + jnp.minimum(s, 4)
    nA = jnp.where(s < 4, 32, 31)
    run_phase(c, gA, nA)
    plsc.subcore_barrier()
    writeout(c)
    plsc.subcore_barrier()
    zero_acc()
    plsc.subcore_barrier()
    gB = c * 250 + 15 * s + jnp.minimum(s, 10)
    nB = jnp.where(s < 10, 16, 15)
    run_phase(2, gB, nB)
    plsc.subcore_barrier()
    writeout(2 + c)


def _sc_segment_sum_raw(Mp, receivers):
    mesh = plsc.VectorSubcoreMesh(core_axis_name="c", subcore_axis_name="s")
    f = pl.kernel(
        _sc_scatter_body,
        out_type=jax.ShapeDtypeStruct((4, NROW, 128), jnp.float32),
        mesh=mesh,
        scratch_types=[
            pltpu.VMEM((CG, 128), jnp.float32),
            pltpu.VMEM((16, CG), jnp.int32),
            pltpu.VMEM_SHARED((NROW, 128), jnp.float32),
            pltpu.SemaphoreType.DMA,
            pltpu.SemaphoreType.DMA,
        ],
    )
    return f(Mp, receivers.astype(jnp.int32).reshape(E // CG, CG))


BN = 2000
_INV = 1.0 / float(_np.sqrt(AVG_NEIGH))

_PERM = _np.arange(D_OUT)
for _k in range(3):
    for _c in range(C1):
        _PERM[LAT + _k * C1 + _c] = LAT + _c * 3 + _k
for _k in range(5):
    for _c in range(C2):
        _PERM[LAT + 3 * C1 + _k * C2 + _c] = LAT + 3 * C1 + _c * 5 + _k


def _node_kernel(a0_ref, a1_ref, a2a_ref, a2b_ref, nf_ref, ws_ref, wf_ref,
                 ne_ref, fl_ref):
    a0 = a0_ref[...] * _INV
    nf = nf_ref[...]
    scal = _silu(a0 + jnp.dot(nf, ws_ref[...],
                              preferred_element_type=jnp.float32))
    g1 = jax.nn.sigmoid(scal[:, :C1])
    g2 = jax.nn.sigmoid(scal[:, C1:C1 + C2])
    a1 = a1_ref[...] * _INV
    a2 = (a2a_ref[...] + a2b_ref[...]) * _INV
    parts = [scal]
    for k in range(3):
        parts.append(a1[:, 32 * k:32 * (k + 1)] * g1)
    for k in range(2):
        parts.append(a1[:, 96 + 16 * k:96 + 16 * (k + 1)] * g2)
    for k in range(3):
        parts.append(a2[:, 16 * k:16 * (k + 1)] * g2)
    ne = jnp.concatenate(parts, axis=-1)
    ne_ref[...] = ne
    fl_ref[...] = jnp.dot(ne, wf_ref[...],
                          preferred_element_type=jnp.float32)


def _node_stage(a0, a1, a2a, a2b, nf, W_self, W_focus_p):
    grid = (N // BN,)
    return pl.pallas_call(
        _node_kernel,
        grid=grid,
        in_specs=[
            pl.BlockSpec((BN, 128), lambda i: (i, 0)),
            pl.BlockSpec((BN, 128), lambda i: (i, 0)),
            pl.BlockSpec((BN, 128), lambda i: (i, 0)),
            pl.BlockSpec((BN, 128), lambda i: (i, 0)),
            pl.BlockSpec((BN, LAT), lambda i: (i, 0)),
            pl.BlockSpec((LAT, LAT), lambda i: (0, 0)),
            pl.BlockSpec((D_OUT, 1), lambda i: (0, 0)),
        ],
        out_specs=[
            pl.BlockSpec((BN, D_OUT), lambda i: (i, 0)),
            pl.BlockSpec((BN, 1), lambda i: (i, 0)),
        ],
        out_shape=[
            jax.ShapeDtypeStruct((N, D_OUT), jnp.float32),
            jax.ShapeDtypeStruct((N, 1), jnp.float32),
        ],
    )(a0, a1, a2a, a2b, nf, W_self, W_focus_p)


def _heads_kernel(fidx_ref, ts_ref, ne_ref, se_ref, wsp_ref, bsp_ref,
                  wpa_ref, wpb_ref, bp_ref, sl_ref, pc_ref):
    tf = ne_ref[0]
    ts = se_ref[0]
    sl_ref[0] = jnp.dot(tf, wsp_ref[...],
                        preferred_element_type=jnp.float32) + bsp_ref[...]
    pc_ref[0] = (jnp.dot(tf, wpa_ref[...],
                         preferred_element_type=jnp.float32)
                 + jnp.dot(ts, wpb_ref[...],
                           preferred_element_type=jnp.float32)
                 + bp_ref[...])


def _heads(ne, species_embed, first_idx, target_species, W_species_p,
           b_species, W_pos_a, W_pos_b, b_pos, B):
    grid_spec = pltpu.PrefetchScalarGridSpec(
        num_scalar_prefetch=2,
        grid=(B,),
        in_specs=[
            pl.BlockSpec((1, 1, D_OUT), lambda b, fidx, ts: (fidx[b], 0, 0)),
            pl.BlockSpec((1, 1, LAT), lambda b, fidx, ts: (ts[b], 0, 0)),
            pl.BlockSpec((D_OUT, 8), lambda b, fidx, ts: (0, 0)),
            pl.BlockSpec((1, 8), lambda b, fidx, ts: (0, 0)),
            pl.BlockSpec((D_OUT, N_RADII * IRREPS_DIM),
                         lambda b, fidx, ts: (0, 0)),
            pl.BlockSpec((LAT, N_RADII * IRREPS_DIM),
                         lambda b, fidx, ts: (0, 0)),
            pl.BlockSpec((1, N_RADII * IRREPS_DIM),
                         lambda b, fidx, ts: (0, 0)),
        ],
        out_specs=[
            pl.BlockSpec((1, 1, 8), lambda b, fidx, ts: (b, 0, 0)),
            pl.BlockSpec((1, 1, N_RADII * IRREPS_DIM),
                         lambda b, fidx, ts: (b, 0, 0)),
        ],
    )
    return pl.pallas_call(
        _heads_kernel,
        grid_spec=grid_spec,
        out_shape=[
            jax.ShapeDtypeStruct((B, 1, 8), jnp.float32),
            jax.ShapeDtypeStruct((B, 1, N_RADII * IRREPS_DIM), jnp.float32),
        ],
    )(first_idx, target_species, ne.reshape(N, 1, D_OUT),
      species_embed.reshape(-1, 1, LAT), W_species_p, b_species,
      W_pos_a, W_pos_b, b_pos)


def kernel(positions, species, senders, receivers, n_node, target_species,
           species_embed, node_embed, W_r1, b_r1, W_r2, W_self,
           W_focus, b_focus, W_species, b_species, W_pos, b_pos):
    B = n_node.shape[0]
    vecT = (positions[receivers] - positions[senders]).T
    spc2 = jnp.pad(species, (0, NROW - N)).astype(jnp.int32).reshape(
        NROW // 40, 40)
    send2 = senders.astype(jnp.int32).reshape(E // 40, 40)
    nf_pad, sf = _sc_gather(node_embed, spc2, send2)
    M = _edge_messages(vecT, sf, W_r1, b_r1, W_r2)
    out4 = _sc_segment_sum_raw(M, receivers)
    perm = jnp.asarray(_PERM)
    W_focus_p = W_focus[perm]
    ne, fl = _node_stage(out4[0, :N], out4[1, :N], out4[2, :N], out4[3, :N],
                         nf_pad[:N], W_self, W_focus_p)
    focus_logits = fl[:, 0] + b_focus[0]
    first_idx = jnp.concatenate(
        [jnp.zeros((1,), jnp.int32), jnp.cumsum(n_node)[:-1].astype(jnp.int32)])
    W_species_p = jnp.zeros((D_OUT, 8), jnp.float32).at[:, :5].set(
        W_species[perm])
    b_species_p = jnp.zeros((1, 8), jnp.float32).at[0, :5].set(b_species)
    W_pos_a = W_pos[:D_OUT][perm]
    W_pos_b = W_pos[D_OUT:]
    sl, pc = _heads(ne, species_embed, first_idx,
                    target_species.astype(jnp.int32), W_species_p,
                    b_species_p, W_pos_a, W_pos_b, b_pos[None, :], B)
    species_logits = sl[:, 0, :5]
    position_coeffs = pc.reshape(B, N_RADII, IRREPS_DIM)
    return focus_logits, species_logits, position_coeffs


# speedup vs baseline: 25.4042x; 1.1870x over previous
import functools

import jax
import jax.numpy as jnp
from jax import lax
from jax.experimental import pallas as pl
from jax.experimental.pallas import tpu as pltpu
from jax.experimental.pallas import tpu_sc as plsc

N = 10000
E = 320000
LAT = 128
NRB = 8
RMAX = 2.0
C1 = 32
C2 = 16
AVG_NEIGH = 32.0
N_RADII = 64
IRREPS_DIM = 36
D_OUT = LAT + C1 * 3 + C2 * 5

BE = 2560


def _silu(x):
    return x * jax.nn.sigmoid(x)


import numpy as _np

_W1P_PAD = 16
_P1 = _np.zeros((16, 128), _np.float32)
for _k in range(3):
    _P1[8 + _k, 32 * _k:32 * (_k + 1)] = 1.0
for _k in range(2):
    _P1[11 + _k, 96 + 16 * _k:96 + 16 * (_k + 1)] = 1.0
_P2 = _np.zeros((16, 128), _np.float32)
for _k in range(3):
    _P2[13 + _k, 16 * _k:16 * (_k + 1)] = 1.0
_Q = _np.zeros((128, 256), _np.float32)
for _k in range(3):
    for _j in range(32):
        _Q[_j, 32 * _k + _j] = 1.0
for _k in range(2):
    for _j in range(16):
        _Q[_j, 96 + 16 * _k + _j] = 1.0
for _k in range(3):
    for _j in range(16):
        _Q[_j, 128 + 16 * _k + _j] = 1.0


def _expand_weights(W_r1, W_r2):
    W1p = jnp.zeros((16, 64), jnp.float32).at[:NRB].set(W_r1)
    w1 = W_r2[:, LAT:LAT + C1]
    w2 = W_r2[:, LAT + C1:]
    W2exp = jnp.concatenate(
        [W_r2[:, :LAT], w1, w1, w1, w2, w2, w2, w2, w2,
         jnp.zeros((64, 80), jnp.float32)], axis=1)
    return W1p, W2exp


def _edge_kernel(vecT_ref, sf_ref, w1p_ref, br1_ref, w2e_ref, p1_ref,
                 p2_ref, q_ref, m_ref):
    vx = vecT_ref[0, :]
    vy = vecT_ref[1, :]
    vz = vecT_ref[2, :]
    sf = sf_ref[...]
    d2 = vx * vx + vy * vy + vz * vz
    d = jnp.sqrt(d2)
    inv = 1.0 / (d + 1e-9)
    th = (jnp.pi / RMAX) * d
    s1 = jnp.sin(th)
    c1 = jnp.cos(th)
    env = jnp.where(d <= RMAX, 0.5 * (c1 + 1.0), 0.0)
    g = jnp.sqrt(2.0 / RMAX) * env * inv
    two_c1 = 2.0 * c1
    cols = [s1 * g]
    s_prev, s_cur = jnp.zeros_like(s1), s1
    for _ in range(NRB - 1):
        s_prev, s_cur = s_cur, two_c1 * s_cur - s_prev
        cols.append(s_cur * g)
    ux = vx * inv
    uy = vy * inv
    uz = vz * inv
    s3 = jnp.sqrt(3.0)
    cols += [ux, uy, uz,
             s3 * ux * uy, s3 * uy * uz, 0.5 * (3.0 * uz * uz - 1.0),
             s3 * ux * uz, 0.5 * s3 * (ux * ux - uy * uy)]
    RU = jnp.stack(cols, axis=-1)
    h = _silu(jnp.dot(RU, w1p_ref[...], preferred_element_type=jnp.float32)
              + br1_ref[...])
    wb = jnp.dot(h, w2e_ref[...], preferred_element_type=jnp.float32)
    S1 = jnp.dot(RU, p1_ref[...], preferred_element_type=jnp.float32)
    S2 = jnp.dot(RU, p2_ref[...], preferred_element_type=jnp.float32)
    sft = jnp.dot(sf, q_ref[...], preferred_element_type=jnp.float32)
    m_ref[0] = sf * wb[:, :128]
    m_ref[1] = S1 * sft[:, :128] * wb[:, 128:256]
    m_ref[2] = S2 * sft[:, 128:256] * wb[:, 256:384]


def _edge_messages(vecT, sf, W_r1, b_r1, W_r2):
    W1p, W2exp = _expand_weights(W_r1, W_r2)
    grid = (E // BE,)
    return pl.pallas_call(
        _edge_kernel,
        grid=grid,
        in_specs=[
            pl.BlockSpec((3, BE), lambda i: (0, i)),
            pl.BlockSpec((BE, LAT), lambda i: (i, 0)),
            pl.BlockSpec((16, 64), lambda i: (0, 0)),
            pl.BlockSpec((64,), lambda i: (0,)),
            pl.BlockSpec((64, 384), lambda i: (0, 0)),
            pl.BlockSpec((16, 128), lambda i: (0, 0)),
            pl.BlockSpec((16, 128), lambda i: (0, 0)),
            pl.BlockSpec((128, 256), lambda i: (0, 0)),
        ],
        out_specs=pl.BlockSpec((3, BE, 128), lambda i: (0, i, 0)),
        out_shape=jax.ShapeDtypeStruct((3, E, 128), jnp.float32),
    )(vecT, sf, W1p, b_r1, W2exp, jnp.asarray(_P1), jnp.asarray(_P2),
      jnp.asarray(_Q))


def _sc_gather_body(emb_hbm, spc2_hbm, send2_hbm, nf_hbm, sf_hbm,
                    ibuf, gbuf, sem):
    c = lax.axis_index("c")
    s = lax.axis_index("s")

    pltpu.sync_copy(spc2_hbm.at[pl.ds(s * 16, 16)], ibuf)

    def nfrow(j, _):
        pltpu.async_copy(emb_hbm.at[ibuf.at[j]], gbuf, sem).wait()
        pltpu.sync_copy(gbuf, nf_hbm.at[pl.ds((s * 16 + j) * 40, 40)])
        return 0

    lax.fori_loop(0, 16, nfrow, 0)
    plsc.subcore_barrier()

    wid = s * 2 + c
    nblk = 31 + (wid < 8).astype(jnp.int32)

    def blk(i, _):
        r0 = (wid + 32 * i) * 8
        pltpu.sync_copy(send2_hbm.at[pl.ds(r0, 8)], ibuf.at[pl.ds(0, 8)])

        def row(j, _):
            pltpu.async_copy(nf_hbm.at[ibuf.at[j]], gbuf, sem).wait()
            pltpu.sync_copy(gbuf, sf_hbm.at[pl.ds((r0 + j) * 40, 40)])
            return 0

        lax.fori_loop(0, 8, row, 0)
        return 0

    lax.fori_loop(0, nblk, blk, 0)


def _sc_gather(node_embed, spc2, send2):
    mesh = plsc.VectorSubcoreMesh(core_axis_name="c", subcore_axis_name="s")
    f = pl.kernel(
        _sc_gather_body,
        out_type=(jax.ShapeDtypeStruct((NROW, LAT), jnp.float32),
                  jax.ShapeDtypeStruct((E, LAT), jnp.float32)),
        mesh=mesh,
        scratch_types=[
            pltpu.VMEM((16, 40), jnp.int32),
            pltpu.VMEM((40, LAT), jnp.float32),
            pltpu.SemaphoreType.DMA,
        ],
    )
    return f(node_embed, spc2, send2)


CG = 80
NB = 4
NROW = 10240


def _sc_scatter_body(mp_hbm, recv2_hbm, out_hbm, mbuf, ibuf, acc,
                     g0, g1, g2, g3, s0, s1, s2, s3):
    c = lax.axis_index("c")
    s = lax.axis_index("s")
    gsems = [g0, g1, g2, g3]
    ssems = [s0, s1, s2, s3]
    zeros = jnp.zeros((16,), jnp.float32)
    trash_v = jnp.full((16,), NROW, jnp.int32)

    def zero_mbuf(rr, _):
        for b in range(NB):
            for f in range(8):
                mbuf[b, rr, pl.ds(f * 16, 16)] = zeros
        return 0

    def fill_ibuf(rr, _):
        for p in range(2):
            for f in range(CG // 16):
                ibuf[p, rr, pl.ds(f * 16, 16)] = trash_v
        return 0

    def zero_acc():
        lax.fori_loop(0, CG, zero_mbuf, 0)
        lax.fori_loop(0, 16, fill_ibuf, 0)

        def zd(q, _):
            pltpu.sync_copy(mbuf.at[0], acc.at[pl.ds(s * 640 + q * CG, CG)])
            return 0

        lax.fori_loop(0, 640 // CG, zd, 0)

    def drain(sem):
        pltpu.make_async_copy(mp_hbm.at[0, pl.ds(0, CG)], mbuf.at[0],
                              sem).wait()

    def run_phase(pidx, grow0, ngrp):
        pltpu.async_copy(mp_hbm.at[pidx, pl.ds(0, CG)], mbuf.at[NB - 1],
                         gsems[NB - 1])

        def group(g, _):
            par = g % 2
            parp = 1 - par
            pltpu.sync_copy(recv2_hbm.at[pl.ds((grow0 + g) * 16, 16)],
                            ibuf.at[par])
            for j in range(16):
                b = j % NB
                bp = (j - 1) % NB
                e0 = ((grow0 + g) * 16 + j) * CG
                pltpu.async_copy(mp_hbm.at[pidx, pl.ds(e0, CG)],
                                 mbuf.at[b], gsems[b])
                drain(gsems[bp])
                jp = (j - 1) % 16
                pj = par if j > 0 else parp
                pltpu.async_copy(mbuf.at[bp], acc.at[ibuf.at[pj, jp]],
                                 ssems[bp], add=True)
                drain(ssems[bp])
            return 0

        lax.fori_loop(0, ngrp, group, 0)
        lastpar = (ngrp - 1) % 2
        drain(gsems[NB - 1])
        pltpu.async_copy(mbuf.at[NB - 1], acc.at[ibuf.at[lastpar, 15]],
                         ssems[NB - 1], add=True)
        drain(ssems[NB - 1])

    def writeout(oidx):
        pltpu.sync_copy(acc.at[pl.ds(s * 640, 640)],
                        out_hbm.at[oidx, pl.ds(s * 640, 640)])

    zero_acc()
    plsc.subcore_barrier()
    gA = 15 * s + jnp.minimum(s, 10)
    nA = jnp.where(s < 10, 16, 15)
    run_phase(c, gA, nA)
    plsc.subcore_barrier()
    writeout(c)
    plsc.subcore_barrier()
    zero_acc()
    plsc.subcore_barrier()
    gB = c * 125 + 7 * s + jnp.minimum(s, 13)
    nB = jnp.where(s < 13, 8, 7)
    run_phase(2, gB, nB)
    plsc.subcore_barrier()
    writeout(2 + c)


def _sc_segment_sum_raw(Mp, receivers):
    mesh = plsc.VectorSubcoreMesh(core_axis_name="c", subcore_axis_name="s")
    f = pl.kernel(
        _sc_scatter_body,
        out_type=jax.ShapeDtypeStruct((4, NROW, 128), jnp.float32),
        mesh=mesh,
        scratch_types=[
            pltpu.VMEM((NB, CG, 128), jnp.float32),
            pltpu.VMEM((2, 16, CG), jnp.int32),
            pltpu.VMEM_SHARED((NROW + 8, 128), jnp.float32),
            pltpu.SemaphoreType.DMA,
            pltpu.SemaphoreType.DMA,
            pltpu.SemaphoreType.DMA,
            pltpu.SemaphoreType.DMA,
            pltpu.SemaphoreType.DMA,
            pltpu.SemaphoreType.DMA,
            pltpu.SemaphoreType.DMA,
            pltpu.SemaphoreType.DMA,
        ],
    )
    return f(Mp, receivers.astype(jnp.int32).reshape(E // CG, CG))


BN = 2000
_INV = 1.0 / float(_np.sqrt(AVG_NEIGH))

_PERM = _np.arange(D_OUT)
for _k in range(3):
    for _c in range(C1):
        _PERM[LAT + _k * C1 + _c] = LAT + _c * 3 + _k
for _k in range(5):
    for _c in range(C2):
        _PERM[LAT + 3 * C1 + _k * C2 + _c] = LAT + 3 * C1 + _c * 5 + _k


def _node_kernel(a0_ref, a1_ref, a2a_ref, a2b_ref, nf_ref, ws_ref, wf_ref,
                 ne_ref, fl_ref):
    a0 = a0_ref[...] * _INV
    nf = nf_ref[...]
    scal = _silu(a0 + jnp.dot(nf, ws_ref[...],
                              preferred_element_type=jnp.float32))
    g1 = jax.nn.sigmoid(scal[:, :C1])
    g2 = jax.nn.sigmoid(scal[:, C1:C1 + C2])
    a1 = a1_ref[...] * _INV
    a2 = (a2a_ref[...] + a2b_ref[...]) * _INV
    parts = [scal]
    for k in range(3):
        parts.append(a1[:, 32 * k:32 * (k + 1)] * g1)
    for k in range(2):
        parts.append(a1[:, 96 + 16 * k:96 + 16 * (k + 1)] * g2)
    for k in range(3):
        parts.append(a2[:, 16 * k:16 * (k + 1)] * g2)
    ne = jnp.concatenate(parts, axis=-1)
    ne_ref[...] = ne
    fl_ref[...] = jnp.dot(ne, wf_ref[...],
                          preferred_element_type=jnp.float32)


def _node_stage(a0, a1, a2a, a2b, nf, W_self, W_focus_p):
    grid = (N // BN,)
    return pl.pallas_call(
        _node_kernel,
        grid=grid,
        in_specs=[
            pl.BlockSpec((BN, 128), lambda i: (i, 0)),
            pl.BlockSpec((BN, 128), lambda i: (i, 0)),
            pl.BlockSpec((BN, 128), lambda i: (i, 0)),
            pl.BlockSpec((BN, 128), lambda i: (i, 0)),
            pl.BlockSpec((BN, LAT), lambda i: (i, 0)),
            pl.BlockSpec((LAT, LAT), lambda i: (0, 0)),
            pl.BlockSpec((D_OUT, 1), lambda i: (0, 0)),
        ],
        out_specs=[
            pl.BlockSpec((BN, D_OUT), lambda i: (i, 0)),
            pl.BlockSpec((BN, 1), lambda i: (i, 0)),
        ],
        out_shape=[
            jax.ShapeDtypeStruct((N, D_OUT), jnp.float32),
            jax.ShapeDtypeStruct((N, 1), jnp.float32),
        ],
    )(a0, a1, a2a, a2b, nf, W_self, W_focus_p)


def _heads_kernel(fidx_ref, ts_ref, ne_ref, se_ref, wsp_ref, bsp_ref,
                  wpa_ref, wpb_ref, bp_ref, sl_ref, pc_ref):
    tf = ne_ref[0]
    ts = se_ref[0]
    sl_ref[0] = jnp.dot(tf, wsp_ref[...],
                        preferred_element_type=jnp.float32) + bsp_ref[...]
    pc_ref[0] = (jnp.dot(tf, wpa_ref[...],
                         preferred_element_type=jnp.float32)
                 + jnp.dot(ts, wpb_ref[...],
                           preferred_element_type=jnp.float32)
                 + bp_ref[...])


def _heads(ne, species_embed, first_idx, target_species, W_species_p,
           b_species, W_pos_a, W_pos_b, b_pos, B):
    grid_spec = pltpu.PrefetchScalarGridSpec(
        num_scalar_prefetch=2,
        grid=(B,),
        in_specs=[
            pl.BlockSpec((1, 1, D_OUT), lambda b, fidx, ts: (fidx[b], 0, 0)),
            pl.BlockSpec((1, 1, LAT), lambda b, fidx, ts: (ts[b], 0, 0)),
            pl.BlockSpec((D_OUT, 8), lambda b, fidx, ts: (0, 0)),
            pl.BlockSpec((1, 8), lambda b, fidx, ts: (0, 0)),
            pl.BlockSpec((D_OUT, N_RADII * IRREPS_DIM),
                         lambda b, fidx, ts: (0, 0)),
            pl.BlockSpec((LAT, N_RADII * IRREPS_DIM),
                         lambda b, fidx, ts: (0, 0)),
            pl.BlockSpec((1, N_RADII * IRREPS_DIM),
                         lambda b, fidx, ts: (0, 0)),
        ],
        out_specs=[
            pl.BlockSpec((1, 1, 8), lambda b, fidx, ts: (b, 0, 0)),
            pl.BlockSpec((1, 1, N_RADII * IRREPS_DIM),
                         lambda b, fidx, ts: (b, 0, 0)),
        ],
    )
    return pl.pallas_call(
        _heads_kernel,
        grid_spec=grid_spec,
        out_shape=[
            jax.ShapeDtypeStruct((B, 1, 8), jnp.float32),
            jax.ShapeDtypeStruct((B, 1, N_RADII * IRREPS_DIM), jnp.float32),
        ],
    )(first_idx, target_species, ne.reshape(N, 1, D_OUT),
      species_embed.reshape(-1, 1, LAT), W_species_p, b_species,
      W_pos_a, W_pos_b, b_pos)


def kernel(positions, species, senders, receivers, n_node, target_species,
           species_embed, node_embed, W_r1, b_r1, W_r2, W_self,
           W_focus, b_focus, W_species, b_species, W_pos, b_pos):
    B = n_node.shape[0]
    vecT = (positions[receivers] - positions[senders]).T
    spc2 = jnp.pad(species, (0, NROW - N)).astype(jnp.int32).reshape(
        NROW // 40, 40)
    send2 = senders.astype(jnp.int32).reshape(E // 40, 40)
    nf_pad, sf = _sc_gather(node_embed, spc2, send2)
    M = _edge_messages(vecT, sf, W_r1, b_r1, W_r2)
    out4 = _sc_segment_sum_raw(M, receivers)
    perm = jnp.asarray(_PERM)
    W_focus_p = W_focus[perm]
    ne, fl = _node_stage(out4[0, :N], out4[1, :N], out4[2, :N], out4[3, :N],
                         nf_pad[:N], W_self, W_focus_p)
    focus_logits = fl[:, 0] + b_focus[0]
    first_idx = jnp.concatenate(
        [jnp.zeros((1,), jnp.int32), jnp.cumsum(n_node)[:-1].astype(jnp.int32)])
    W_species_p = jnp.zeros((D_OUT, 8), jnp.float32).at[:, :5].set(
        W_species[perm])
    b_species_p = jnp.zeros((1, 8), jnp.float32).at[0, :5].set(b_species)
    W_pos_a = W_pos[:D_OUT][perm]
    W_pos_b = W_pos[D_OUT:]
    sl, pc = _heads(ne, species_embed, first_idx,
                    target_species.astype(jnp.int32), W_species_p,
                    b_species_p, W_pos_a, W_pos_b, b_pos[None, :], B)
    species_logits = sl[:, 0, :5]
    position_coeffs = pc.reshape(B, N_RADII, IRREPS_DIM)
    return focus_logits, species_logits, position_coeffs
